# Initial kernel scaffold; baseline (speedup 1.0000x reference)
import jax
import jax.numpy as jnp
from jax.experimental import pallas as pl


def kernel(sigmas, rgbs, deltas, cu_seqlens, nears, fars):
    raise NotImplementedError("write your pallas kernel here")


# speedup vs baseline: 60.7848x; 60.7848x over previous
import functools

import jax
import jax.numpy as jnp
from jax import lax
from jax.experimental import pallas as pl
from jax.experimental.pallas import tpu as pltpu
from jax.experimental.pallas import tpu_sc as plsc

_TOTAL = 1048576
_NRAYS = 8192
_L = 16
_S = 4096


@functools.cache
def _build(nc: int, ns: int):
    nw = nc * ns
    rpw = _NRAYS // nw
    mesh = plsc.VectorSubcoreMesh(
        core_axis_name="c", subcore_axis_name="s",
        num_cores=nc, num_subcores=ns)

    def body(sig_hbm, rgb_hbm, del_hbm, cu_hbm, near_hbm, far_hbm,
             img_out, dep_out, ws_out,
             cu_v, near_v, far_v, sig_sl, del_sl, rgb_sl,
             img_st, dep_st, ws_st, dma_sem):
        w = lax.axis_index("s") * nc + lax.axis_index("c")
        r0 = pl.multiple_of(w * rpw, 8)
        pltpu.sync_copy(cu_hbm.at[pl.ds(r0, rpw + 24)], cu_v)
        pltpu.sync_copy(near_hbm.at[pl.ds(r0, rpw)], near_v.at[pl.ds(0, rpw)])
        pltpu.sync_copy(far_hbm.at[pl.ds(r0, rpw)], far_v.at[pl.ds(0, rpw)])

        iota = lax.iota(jnp.int32, _L)
        zeros_i = jnp.zeros((_L,), jnp.int32)
        ones_i = zeros_i + 1
        twos_i = zeros_i + 2
        zf = jnp.zeros((_L,), jnp.float32)
        lane0 = iota == 0
        first3 = iota < 3

        def ray_body(r, cur_slab):
            se = cu_v[pl.ds(r, _L)]
            start = se[0]
            end = se[1]
            start_al = lax.div(start, _L) * _L
            ntrips = lax.div(end - start_al + (_L - 1), _L)

            def chunk(j, st):
                c, a0, a1, a2, a3, a4, cur = st
                gp = start_al + j * _L
                sn = lax.div(gp, _S)
                need = sn != cur

                @pl.when(need)
                def _load_slab():
                    sb = pl.multiple_of(sn * _S, _S)
                    sb2 = pl.multiple_of(sn * (2 * _S), 8)
                    sb3 = pl.multiple_of(sn * (3 * _S), 8)
                    h1 = pltpu.async_copy(sig_hbm.at[pl.ds(sb, _S)],
                                          sig_sl, dma_sem)
                    h2 = pltpu.async_copy(del_hbm.at[pl.ds(sb2, 2 * _S)],
                                          del_sl, dma_sem)
                    h3 = pltpu.async_copy(rgb_hbm.at[pl.ds(sb3, 3 * _S)],
                                          rgb_sl, dma_sem)
                    h1.wait()
                    h2.wait()
                    h3.wait()

                cur = jnp.where(need, sn, cur)
                gidx = gp + iota
                valid = (gidx >= start) & (gidx < end)
                idxc = gp - sn * _S + iota
                idx2 = idxc * 2
                idx3 = idxc * 3
                sigv = plsc.load_gather(sig_sl, [idxc])
                dtv = plsc.load_gather(del_sl, [idx2])
                tv = plsc.load_gather(del_sl, [idx2 + 1])
                l = jnp.where(valid, -sigv * dtv, 0.0)
                alc = jnp.exp(l)
                svec = plsc.cumsum(l)
                tr = jnp.exp(c + svec - l)
                wgt = (1.0 - alc) * tr
                c = c + jnp.sum(l)
                rv = plsc.load_gather(rgb_sl, [idx3])
                gv = plsc.load_gather(rgb_sl, [idx3 + 1])
                bv = plsc.load_gather(rgb_sl, [idx3 + 2])
                a0 = a0 + wgt
                a1 = a1 + jnp.where(valid, wgt * tv, 0.0)
                a2 = a2 + jnp.where(valid, wgt * rv, 0.0)
                a3 = a3 + jnp.where(valid, wgt * gv, 0.0)
                a4 = a4 + jnp.where(valid, wgt * bv, 0.0)
                return (c, a0, a1, a2, a3, a4, cur)

            _, a0, a1, a2, a3, a4, cur_slab = lax.fori_loop(
                0, ntrips, chunk,
                (jnp.float32(0.0), zf, zf, zf, zf, zf, cur_slab))

            ws_s = jnp.sum(a0)
            near_vec = near_v[pl.ds(r, _L)]
            far_vec = far_v[pl.ds(r, _L)]
            dep_vec = (jnp.maximum(jnp.full((_L,), jnp.sum(a1)) - near_vec,
                                   0.0)
                       / (far_vec - near_vec))
            bg = 1.0 - ws_s
            plsc.store_scatter(ws_st, [r + zeros_i],
                               jnp.full((_L,), ws_s), mask=lane0)
            plsc.store_scatter(dep_st, [r + zeros_i], dep_vec, mask=lane0)
            rgb_fin = jnp.where(
                iota == 0, jnp.sum(a2) + bg,
                jnp.where(iota == 1, jnp.sum(a3) + bg, jnp.sum(a4) + bg))
            plsc.store_scatter(img_st, [3 * r + iota], rgb_fin, mask=first3)
            return cur_slab

        lax.fori_loop(0, rpw, ray_body, jnp.int32(-1))

        pltpu.sync_copy(ws_st, ws_out.at[pl.ds(r0, rpw)])
        pltpu.sync_copy(dep_st, dep_out.at[pl.ds(r0, rpw)])
        pltpu.sync_copy(img_st,
                        img_out.at[pl.ds(pl.multiple_of(3 * r0, 8), 3 * rpw)])

    return pl.kernel(
        body,
        out_type=(
            jax.ShapeDtypeStruct((3 * _NRAYS,), jnp.float32),
            jax.ShapeDtypeStruct((_NRAYS,), jnp.float32),
            jax.ShapeDtypeStruct((_NRAYS,), jnp.float32),
        ),
        mesh=mesh,
        compiler_params=pltpu.CompilerParams(needs_layout_passes=False),
        scratch_types=[
            pltpu.VMEM((rpw + 24,), jnp.int32),
            pltpu.VMEM((rpw + 16,), jnp.float32),
            pltpu.VMEM((rpw + 16,), jnp.float32),
            pltpu.VMEM((_S,), jnp.float32),
            pltpu.VMEM((2 * _S,), jnp.float32),
            pltpu.VMEM((3 * _S,), jnp.float32),
            pltpu.VMEM((3 * rpw,), jnp.float32),
            pltpu.VMEM((rpw,), jnp.float32),
            pltpu.VMEM((rpw,), jnp.float32),
            pltpu.SemaphoreType.DMA,
        ],
    )


def kernel(sigmas, rgbs, deltas, cu_seqlens, nears, fars):
    try:
        info = plsc.get_sparse_core_info()
        nc, ns = info.num_cores, info.num_subcores
    except Exception:
        nc, ns = 2, 16
    rpw = _NRAYS // (nc * ns)
    k = _build(nc, ns)
    cu_pad = jnp.concatenate(
        [cu_seqlens.astype(jnp.int32),
         jnp.full((rpw + 24,), _TOTAL, dtype=jnp.int32)])
    img_flat, dep, ws = k(sigmas, rgbs.reshape(-1), deltas.reshape(-1),
                          cu_pad, nears, fars)
    return img_flat.reshape(_NRAYS, 3), dep, ws
